# baseline (speedup 1.0000x reference)
import numpy as np
import jax
import jax.numpy as jnp
from jax.experimental import pallas as pl
from jax.experimental.pallas import tpu as pltpu

_M = 1048576
_T = _M // 128
_R = 512
_GRID = _T // _R

_S_np = np.zeros((3, 128, 384), dtype=np.float32)
for _j in range(3):
    _S_np[_j, np.arange(128), 3 * np.arange(128) + _j] = 1.0


def _dot(p, s):
    return jax.lax.dot_general(p, s, (((1,), (0,)), ((), ())),
                               preferred_element_type=jnp.float32)


def _body(x_ref, s_ref, a_ref, index_ref, o_ref):
    b = pl.program_id(0)
    acc = _dot(x_ref[0], s_ref[0])
    acc += _dot(x_ref[1], s_ref[1])
    acc += _dot(x_ref[2], s_ref[2])
    o_ref[...] = acc

    row_lo = b * _R
    for k in range(3):
        idx = index_ref[k]
        t = idx // 128
        lane0 = 3 * (idx % 128)
        ltr = t - row_lo
        in_blk = jnp.logical_and(t >= row_lo, t < row_lo + _R)

        @pl.when(jnp.logical_and(in_blk, ltr < 8))
        def _():
            rows = jax.lax.broadcasted_iota(jnp.int32, (8, 384), 0)
            lanes = jax.lax.broadcasted_iota(jnp.int32, (8, 384), 1)
            upd = jnp.zeros((8, 384), jnp.float32)
            for j in range(3):
                hit = jnp.logical_and(rows == ltr, lanes == lane0 + j)
                upd += jnp.where(hit, a_ref[k, j], 0.0)
            o_ref[0:8, :] += upd

        @pl.when(jnp.logical_and(in_blk, ltr >= 8))
        def _():
            rows = jax.lax.broadcasted_iota(jnp.int32, (_R, 384), 0)
            lanes = jax.lax.broadcasted_iota(jnp.int32, (_R, 384), 1)
            upd = jnp.zeros((_R, 384), jnp.float32)
            for j in range(3):
                hit = jnp.logical_and(rows == ltr, lanes == lane0 + j)
                upd += jnp.where(hit, a_ref[k, j], 0.0)
            o_ref[...] += upd


def kernel(x, a, index):
    x3 = x.reshape(3, _T, 128)
    out = pl.pallas_call(
        _body,
        grid=(_GRID,),
        in_specs=[
            pl.BlockSpec((3, _R, 128), lambda i: (0, i, 0)),
            pl.BlockSpec((3, 128, 384), lambda i: (0, 0, 0)),
            pl.BlockSpec(memory_space=pltpu.SMEM),
            pl.BlockSpec(memory_space=pltpu.SMEM),
        ],
        out_specs=pl.BlockSpec((_R, 384), lambda i: (i, 0)),
        out_shape=jax.ShapeDtypeStruct((_T, 384), jnp.float32),
    )(x3, jnp.asarray(_S_np), a, index.astype(jnp.int32))
    return out.reshape(_M, 3)


# speedup vs baseline: 17.8621x; 17.8621x over previous
import jax
import jax.numpy as jnp
from jax.experimental import pallas as pl
from jax.experimental.pallas import tpu as pltpu

_M = 1048576
_C = 16384
_GRID = _M // _C


def _body(x_ref, a_ref, index_ref, o_ref):
    b = pl.program_id(0)
    o_ref[...] = x_ref[...]

    col_lo = b * _C
    for k in range(3):
        idx = index_ref[k]
        rel = idx - col_lo
        in_blk = jnp.logical_and(idx >= col_lo, idx < col_lo + _C)

        @pl.when(in_blk)
        def _():
            win = pl.multiple_of((rel // 128) * 128, 128)
            lane = rel - (rel // 128) * 128
            lanes = jax.lax.broadcasted_iota(jnp.int32, (1, 128), 1)
            hit = lanes == lane
            for j in range(3):
                sub = o_ref[j:j + 1, pl.ds(win, 128)]
                upd = jnp.where(hit, a_ref[k, j], 0.0)
                o_ref[j:j + 1, pl.ds(win, 128)] = sub + upd


def kernel(x, a, index):
    out = pl.pallas_call(
        _body,
        grid=(_GRID,),
        in_specs=[
            pl.BlockSpec((3, _C), lambda i: (0, i)),
            pl.BlockSpec(memory_space=pltpu.SMEM),
            pl.BlockSpec(memory_space=pltpu.SMEM),
        ],
        out_specs=pl.BlockSpec((3, _C), lambda i: (0, i)),
        out_shape=jax.ShapeDtypeStruct((3, _M), jnp.float32),
    )(x, a, index.astype(jnp.int32))
    return jnp.transpose(out, (1, 0))
